# baseline (speedup 1.0000x reference)
import functools

import jax
import jax.numpy as jnp
from jax import lax
from jax.experimental import pallas as pl
from jax.experimental.pallas import tpu as pltpu
from jax.experimental.pallas import tpu_sc as plsc

_BATCH = 16384
_D = 64
_NC = 2
_NS = 16
_NW = _NC * _NS
_BPW = _BATCH // _NW
_CHUNK = 128
_NCH = _BPW // _CHUNK
_G = _BPW // 16


def _mf_body(uid_hbm, iid_hbm, ut_hbm, it_hbm, ub_hbm, ib_hbm, gb_hbm,
             out_hbm,
             uid_v, iid_v, uid_f, iid_f, u_blk, i_blk, u_t, i_t, ub_v, ib_v,
             gb_v, out_v, sem, bsem):
  wid = lax.axis_index("s") * _NC + lax.axis_index("c")
  base = wid * _BPW

  pltpu.sync_copy(uid_hbm.at[pl.ds(base, _BPW)], uid_f)
  pltpu.sync_copy(iid_hbm.at[pl.ds(base, _BPW)], iid_f)
  for c in range(_NCH):
    pltpu.sync_copy(uid_hbm.at[pl.ds(base + c * _CHUNK, _CHUNK)], uid_v.at[c])
    pltpu.sync_copy(iid_hbm.at[pl.ds(base + c * _CHUNK, _CHUNK)], iid_v.at[c])
  pltpu.sync_copy(gb_hbm, gb_v)

  bcopies = []
  for c in range(_NCH):
    sl = pl.ds(c * _CHUNK, _CHUNK)
    bcopies.append(pltpu.async_copy(ub_hbm.at[uid_v.at[c]], ub_v.at[sl], bsem))
    bcopies.append(pltpu.async_copy(ib_hbm.at[iid_v.at[c]], ib_v.at[sl], bsem))

  lanes = lax.iota(jnp.int32, 16)

  def fire(g, carry):
    rv = uid_f[pl.ds(g * 16, 16)]
    sv = iid_f[pl.ds(g * 16, 16)]
    rb = rv & ~7
    sb = sv & ~7
    copies = []
    for k in range(16):
      copies.append(pltpu.async_copy(
          ut_hbm.at[:, pl.ds(pl.multiple_of(rb[k], 8), 8)], u_blk.at[k], sem))
      copies.append(pltpu.async_copy(
          it_hbm.at[:, pl.ds(pl.multiple_of(sb[k], 8), 8)], i_blk.at[k], sem))
    for cp in copies:
      cp.wait()
    r7 = rv & 7
    s7 = sv & 7
    dst = pl.ds(g * 16, 16)
    for j in range(_D):
      jv = jnp.full((16,), j, jnp.int32)
      u_t[j, dst] = plsc.load_gather(u_blk, [lanes, jv, r7])
      i_t[j, dst] = plsc.load_gather(i_blk, [lanes, jv, s7])
    return carry

  lax.fori_loop(0, _G, fire, 0)
  for cp in bcopies:
    cp.wait()

  gbvec = gb_v[...]

  def chunk(c, carry):
    for s in range(_CHUNK // 16):
      off = c * _CHUNK + s * 16
      sl = pl.ds(off, 16)
      acc = u_t[0, sl] * i_t[0, sl]
      for j in range(1, _D):
        acc = acc + u_t[j, sl] * i_t[j, sl]
      out_v[sl] = acc + ub_v[sl] + ib_v[sl] + gbvec
    return carry

  lax.fori_loop(0, _NCH, chunk, 0)

  pltpu.sync_copy(out_v, out_hbm.at[pl.ds(base, _BPW)])


@jax.jit
def _mf(uid, iid, ut, it, ub, ib, gb):
  mesh = plsc.VectorSubcoreMesh(core_axis_name="c", subcore_axis_name="s")
  f = functools.partial(
      pl.kernel,
      out_type=jax.ShapeDtypeStruct((_BATCH,), jnp.float32),
      mesh=mesh,
      compiler_params=pltpu.CompilerParams(
          needs_layout_passes=False, use_tc_tiling_on_sc=False),
      scratch_types=[
          pltpu.VMEM((_NCH, _CHUNK), jnp.int32),
          pltpu.VMEM((_NCH, _CHUNK), jnp.int32),
          pltpu.VMEM((_BPW,), jnp.int32),
          pltpu.VMEM((_BPW,), jnp.int32),
          pltpu.VMEM((16, _D, 8), jnp.float32),
          pltpu.VMEM((16, _D, 8), jnp.float32),
          pltpu.VMEM((_D, _BPW), jnp.float32),
          pltpu.VMEM((_D, _BPW), jnp.float32),
          pltpu.VMEM((_BPW,), jnp.float32),
          pltpu.VMEM((_BPW,), jnp.float32),
          pltpu.VMEM((16,), jnp.float32),
          pltpu.VMEM((_BPW,), jnp.float32),
          pltpu.SemaphoreType.DMA,
          pltpu.SemaphoreType.DMA,
      ],
  )(_mf_body)
  return f(uid, iid, ut, it, ub, ib, gb)


def kernel(user_ids, item_ids, user_embeddings, item_embeddings, user_bias,
           item_bias, global_bias):
  uid = user_ids.astype(jnp.int32)
  iid = item_ids.astype(jnp.int32)
  ut = user_embeddings.T
  it = item_embeddings.T
  ub = user_bias.reshape(-1)
  ib = item_bias.reshape(-1)
  gb16 = jnp.broadcast_to(global_bias.reshape(-1), (16,))
  return _mf(uid, iid, ut, it, ub, ib, gb16)


# speedup vs baseline: 8.4665x; 8.4665x over previous
import functools

import jax
import jax.numpy as jnp
from jax import lax
from jax.experimental import pallas as pl
from jax.experimental.pallas import tpu as pltpu
from jax.experimental.pallas import tpu_sc as plsc

_BATCH = 16384
_D = 64
_NC = 2
_NS = 16
_NW = _NC * _NS
_BPW = _BATCH // _NW
_CHUNK = 128
_NCH = _BPW // _CHUNK
_G = _BPW // 16

_params = pltpu.CompilerParams(
    needs_layout_passes=False, use_tc_tiling_on_sc=False)
_mesh = plsc.VectorSubcoreMesh(core_axis_name="c", subcore_axis_name="s")


def _gather_body(ids_hbm, tab_hbm, out_hbm,
                 idx_v, q_v, idx_f, rows_v, ext_v, sem):
  wid = lax.axis_index("s") * _NC + lax.axis_index("c")
  base = wid * _BPW

  pltpu.sync_copy(ids_hbm.at[pl.ds(base, _BPW)], idx_f)
  for c in range(_NCH):
    pltpu.sync_copy(ids_hbm.at[pl.ds(base + c * _CHUNK, _CHUNK)], idx_v.at[c])

  for c in range(_NCH):
    for s in range(_CHUNK // 16):
      sl = pl.ds(s * 16, 16)
      q_v[c, sl] = lax.shift_right_logical(idx_v[c, sl], 1)

  copies = []
  for c in range(_NCH):
    copies.append(pltpu.async_copy(
        tab_hbm.at[q_v.at[c]], rows_v.at[pl.ds(c * _CHUNK, _CHUNK)], sem))
  for cp in copies:
    cp.wait()

  lanes = lax.iota(jnp.int32, 16)

  def extract(g, carry):
    rv = idx_f[pl.ds(g * 16, 16)]
    half = (rv & 1) * _D
    slots = g * 16 + lanes
    dst = pl.ds(g * 16, 16)
    for j in range(_D):
      ext_v[j, dst] = plsc.load_gather(rows_v, [slots, half + j])
    return carry

  lax.fori_loop(0, _G, extract, 0)

  for j in range(_D):
    pltpu.sync_copy(ext_v.at[j],
                    out_hbm.at[pl.ds(j * _BATCH + base, _BPW)])


@functools.partial(
    pl.kernel,
    out_type=jax.ShapeDtypeStruct((_D * _BATCH,), jnp.float32),
    mesh=_mesh,
    compiler_params=_params,
    scratch_types=[
        pltpu.VMEM((_NCH, _CHUNK), jnp.int32),
        pltpu.VMEM((_NCH, _CHUNK), jnp.int32),
        pltpu.VMEM((_BPW,), jnp.int32),
        pltpu.VMEM((_BPW, 2 * _D), jnp.float32),
        pltpu.VMEM((_D, _BPW), jnp.float32),
        pltpu.SemaphoreType.DMA,
    ],
)
def _gather(ids_hbm, tab_hbm, out_hbm, *rest):
  _gather_body(ids_hbm, tab_hbm, out_hbm, *rest)


def _dot_body(ug_hbm, ig_hbm, uid_hbm, iid_hbm, ub_hbm, ib_hbm, gb_hbm,
              out_hbm,
              uid_v, iid_v, u_t, i_t, ub_v, ib_v, gb_v, out_v, sem):
  wid = lax.axis_index("s") * _NC + lax.axis_index("c")
  base = wid * _BPW

  for c in range(_NCH):
    pltpu.sync_copy(uid_hbm.at[pl.ds(base + c * _CHUNK, _CHUNK)], uid_v.at[c])
    pltpu.sync_copy(iid_hbm.at[pl.ds(base + c * _CHUNK, _CHUNK)], iid_v.at[c])
  pltpu.sync_copy(gb_hbm, gb_v)

  copies = []
  for j in range(_D):
    copies.append(pltpu.async_copy(
        ug_hbm.at[pl.ds(j * _BATCH + base, _BPW)], u_t.at[j], sem))
    copies.append(pltpu.async_copy(
        ig_hbm.at[pl.ds(j * _BATCH + base, _BPW)], i_t.at[j], sem))
  for c in range(_NCH):
    sl = pl.ds(c * _CHUNK, _CHUNK)
    copies.append(pltpu.async_copy(ub_hbm.at[uid_v.at[c]], ub_v.at[sl], sem))
    copies.append(pltpu.async_copy(ib_hbm.at[iid_v.at[c]], ib_v.at[sl], sem))
  for cp in copies:
    cp.wait()

  gbvec = gb_v[...]

  def chunk(c, carry):
    for s in range(_CHUNK // 16):
      off = c * _CHUNK + s * 16
      sl = pl.ds(off, 16)
      acc = u_t[0, sl] * i_t[0, sl]
      for j in range(1, _D):
        acc = acc + u_t[j, sl] * i_t[j, sl]
      out_v[sl] = acc + ub_v[sl] + ib_v[sl] + gbvec
    return carry

  lax.fori_loop(0, _NCH, chunk, 0)

  pltpu.sync_copy(out_v, out_hbm.at[pl.ds(base, _BPW)])


@functools.partial(
    pl.kernel,
    out_type=jax.ShapeDtypeStruct((_BATCH,), jnp.float32),
    mesh=_mesh,
    compiler_params=_params,
    scratch_types=[
        pltpu.VMEM((_NCH, _CHUNK), jnp.int32),
        pltpu.VMEM((_NCH, _CHUNK), jnp.int32),
        pltpu.VMEM((_D, _BPW), jnp.float32),
        pltpu.VMEM((_D, _BPW), jnp.float32),
        pltpu.VMEM((_BPW,), jnp.float32),
        pltpu.VMEM((_BPW,), jnp.float32),
        pltpu.VMEM((16,), jnp.float32),
        pltpu.VMEM((_BPW,), jnp.float32),
        pltpu.SemaphoreType.DMA,
    ],
)
def _dot(ug, ig, uid, iid, ub, ib, gb, out, *rest):
  _dot_body(ug, ig, uid, iid, ub, ib, gb, out, *rest)


@jax.jit
def _mf(uid, iid, u2, i2, ub, ib, gb16):
  ug = _gather(uid, u2)
  ig = _gather(iid, i2)
  return _dot(ug, ig, uid, iid, ub, ib, gb16)


def kernel(user_ids, item_ids, user_embeddings, item_embeddings, user_bias,
           item_bias, global_bias):
  uid = user_ids.astype(jnp.int32)
  iid = item_ids.astype(jnp.int32)
  u2 = user_embeddings.reshape(500000, 128)
  i2 = item_embeddings.reshape(500000, 128)
  ub = user_bias.reshape(-1)
  ib = item_bias.reshape(-1)
  gb16 = jnp.broadcast_to(global_bias.reshape(-1), (16,))
  return _mf(uid, iid, u2, i2, ub, ib, gb16)


# speedup vs baseline: 8.9897x; 1.0618x over previous
import functools

import jax
import jax.numpy as jnp
from jax import lax
from jax.experimental import pallas as pl
from jax.experimental.pallas import tpu as pltpu
from jax.experimental.pallas import tpu_sc as plsc

_BATCH = 16384
_D = 64
_NC = 2
_NS = 16
_NW = _NC * _NS
_BPW = _BATCH // _NW
_CHUNK = 128
_NCH = _BPW // _CHUNK


def _mf_body(uid_hbm, iid_hbm, uemb_hbm, iemb_hbm, ub_hbm, ib_hbm, gb_hbm,
             out_hbm,
             uid_v, iid_v, urows_v, irows_v, ub_v, ib_v, gb_v, out_v, sem):
  wid = lax.axis_index("s") * _NC + lax.axis_index("c")
  base = wid * _BPW

  for c in range(_NCH):
    pltpu.sync_copy(uid_hbm.at[pl.ds(base + c * _CHUNK, _CHUNK)], uid_v.at[c])
    pltpu.sync_copy(iid_hbm.at[pl.ds(base + c * _CHUNK, _CHUNK)], iid_v.at[c])
  pltpu.sync_copy(gb_hbm, gb_v)

  copies = []
  for c in range(_NCH):
    sl = pl.ds(c * _CHUNK, _CHUNK)
    copies.append(pltpu.async_copy(uemb_hbm.at[uid_v.at[c]], urows_v.at[sl], sem))
    copies.append(pltpu.async_copy(iemb_hbm.at[iid_v.at[c]], irows_v.at[sl], sem))
    copies.append(pltpu.async_copy(ub_hbm.at[uid_v.at[c]], ub_v.at[sl], sem))
    copies.append(pltpu.async_copy(ib_hbm.at[iid_v.at[c]], ib_v.at[sl], sem))
  for cp in copies:
    cp.wait()

  gbvec = gb_v[...]

  def group(g, carry):
    rbase = g * 16
    rows = rbase + lax.iota(jnp.int32, 16)
    acc = jnp.zeros((16,), jnp.float32)
    for j in range(_D):
      cols = jnp.full((16,), j, jnp.int32)
      u = plsc.load_gather(urows_v, [rows, cols])
      w = plsc.load_gather(irows_v, [rows, cols])
      acc = acc + u * w
    res = acc + ub_v[pl.ds(rbase, 16)] + ib_v[pl.ds(rbase, 16)] + gbvec
    out_v[pl.ds(rbase, 16)] = res
    return carry

  lax.fori_loop(0, _BPW // 16, group, 0)

  pltpu.sync_copy(out_v, out_hbm.at[pl.ds(base, _BPW)])


@jax.jit
def _mf(uid, iid, uemb, iemb, ub, ib, gb):
  mesh = plsc.VectorSubcoreMesh(core_axis_name="c", subcore_axis_name="s")
  f = functools.partial(
      pl.kernel,
      out_type=jax.ShapeDtypeStruct((_BATCH,), jnp.float32),
      mesh=mesh,
      compiler_params=pltpu.CompilerParams(
          needs_layout_passes=False, use_tc_tiling_on_sc=False),
      scratch_types=[
          pltpu.VMEM((_NCH, _CHUNK), jnp.int32),
          pltpu.VMEM((_NCH, _CHUNK), jnp.int32),
          pltpu.VMEM((_BPW, _D), jnp.float32),
          pltpu.VMEM((_BPW, _D), jnp.float32),
          pltpu.VMEM((_BPW,), jnp.float32),
          pltpu.VMEM((_BPW,), jnp.float32),
          pltpu.VMEM((16,), jnp.float32),
          pltpu.VMEM((_BPW,), jnp.float32),
          pltpu.SemaphoreType.DMA,
      ],
  )(_mf_body)
  return f(uid, iid, uemb, iemb, ub, ib, gb)


def kernel(user_ids, item_ids, user_embeddings, item_embeddings, user_bias,
           item_bias, global_bias):
  uid = user_ids.astype(jnp.int32)
  iid = item_ids.astype(jnp.int32)
  ub = user_bias.reshape(-1)
  ib = item_bias.reshape(-1)
  gb16 = jnp.broadcast_to(global_bias.reshape(-1), (16,))
  return _mf(uid, iid, user_embeddings, item_embeddings, ub, ib, gb16)
